# Initial kernel scaffold; baseline (speedup 1.0000x reference)
import jax
import jax.numpy as jnp
from jax.experimental import pallas as pl


def kernel(x, position_embedding, position_start):
    raise NotImplementedError("write your pallas kernel here")


# speedup vs baseline: 2.0846x; 2.0846x over previous
import functools

import jax
import jax.numpy as jnp
from jax.experimental import pallas as pl
from jax.experimental.pallas import tpu as pltpu

_BS = 256


def _body(start_ref, pe_ref, x_ref, o_ref):
    i = pl.program_id(0)
    start = start_ref[0]
    row0 = pl.multiple_of(start + i * _BS, 8)
    pe_blk = pe_ref[pl.ds(row0, _BS), :]
    o_ref[...] = x_ref[...] + pe_blk[None, :, :]


@functools.partial(jax.jit, static_argnames=())
def _pe_add(x, position_embedding, start):
    B, S, D = x.shape
    grid = (S // _BS,)
    return pl.pallas_call(
        _body,
        grid_spec=pltpu.PrefetchScalarGridSpec(
            num_scalar_prefetch=1,
            grid=grid,
            in_specs=[
                pl.BlockSpec(position_embedding.shape, lambda i, s_ref: (0, 0)),
                pl.BlockSpec((B, _BS, D), lambda i, s_ref: (0, i, 0)),
            ],
            out_specs=pl.BlockSpec((B, _BS, D), lambda i, s_ref: (0, i, 0)),
        ),
        out_shape=jax.ShapeDtypeStruct(x.shape, x.dtype),
    )(start, position_embedding, x)


def kernel(x, position_embedding, position_start):
    start = jnp.asarray(position_start, jnp.int32).reshape((1,))
    return _pe_add(x, position_embedding, start)


# speedup vs baseline: 2.2033x; 1.0569x over previous
import functools

import jax
import jax.numpy as jnp
from jax.experimental import pallas as pl
from jax.experimental.pallas import tpu as pltpu

_BS = 512


def _body(start_ref, pe_ref, x_ref, o_ref):
    i = pl.program_id(0)
    start = start_ref[0]
    row0 = pl.multiple_of(start + i * _BS, 8)
    pe_blk = pe_ref[pl.ds(row0, _BS), :]
    o_ref[...] = x_ref[...] + pe_blk[None, :, :]


@functools.partial(jax.jit, static_argnames=())
def _pe_add(x, position_embedding, start):
    B, S, D = x.shape
    grid = (S // _BS,)
    return pl.pallas_call(
        _body,
        grid_spec=pltpu.PrefetchScalarGridSpec(
            num_scalar_prefetch=1,
            grid=grid,
            in_specs=[
                pl.BlockSpec(position_embedding.shape, lambda i, s_ref: (0, 0)),
                pl.BlockSpec((B, _BS, D), lambda i, s_ref: (0, i, 0)),
            ],
            out_specs=pl.BlockSpec((B, _BS, D), lambda i, s_ref: (0, i, 0)),
        ),
        out_shape=jax.ShapeDtypeStruct(x.shape, x.dtype),
        compiler_params=pltpu.CompilerParams(
            dimension_semantics=("parallel",),
        ),
    )(start, position_embedding, x)


def kernel(x, position_embedding, position_start):
    start = jnp.asarray(position_start, jnp.int32).reshape((1,))
    return _pe_add(x, position_embedding, start)
